# baseline (speedup 1.0000x reference)
import functools
import math

import jax
import jax.numpy as jnp
from jax import lax
from jax.experimental import pallas as pl
from jax.experimental.pallas import tpu as pltpu
from jax.experimental.pallas import tpu_sc as plsc

D_MODEL = 64
SCALE = math.sqrt(D_MODEL)

_info = plsc.get_sparse_core_info()
NC, NS, L = _info.num_cores, _info.num_subcores, _info.num_lanes
NW = NC * NS


def _make_kernel(B, D, C, NBUF, U):
    per_w = B // NW
    nchunks = per_w // C
    ngroups = nchunks // NBUF
    assert per_w % C == 0 and nchunks % NBUF == 0 and C % U == 0
    assert (C * D) % 128 == 0
    mesh = plsc.VectorSubcoreMesh(core_axis_name="c", subcore_axis_name="s")

    @functools.partial(
        pl.kernel,
        mesh=mesh,
        out_type=jax.ShapeDtypeStruct((B, 2 * D), jnp.float32),
        scratch_types=[
            pltpu.VMEM((NBUF, C), jnp.int32),
            pltpu.VMEM((NBUF, C, 2 * D), jnp.float32),
        ]
        + [pltpu.SemaphoreType.DMA] * (2 * NBUF),
        compiler_params=pltpu.CompilerParams(use_tc_tiling_on_sc=False),
    )
    def k(idx_hbm, lut_hbm, out_hbm, idx_v, rows_v, *sems):
        gsem, osem = sems[:NBUF], sems[NBUF:]
        wid = lax.axis_index("s") * NC + lax.axis_index("c")
        base = wid * per_w

        def scale_chunk(b):
            def body(r0, carry):
                for u in range(U):
                    r = r0 * U + u
                    for j in range(D // L):
                        sl = pl.ds(j * L, L)
                        rows_v[b, r, sl] = rows_v[b, r, sl] * SCALE
                return carry

            lax.fori_loop(0, C // U, body, 0)

        def out_slice(g):
            return out_hbm.at[pl.ds(base + g * C, C), pl.ds(0, D)]

        for b in range(NBUF):
            row0 = base + b * C
            pltpu.sync_copy(idx_hbm.at[pl.ds(row0, C)], idx_v.at[b])
            pltpu.async_copy(lut_hbm.at[idx_v.at[b]], rows_v.at[b], gsem[b])

        def group(gi, carry):
            for b in range(NBUF):
                g = gi * NBUF + b
                row0 = base + g * C
                pltpu.make_async_copy(
                    lut_hbm.at[idx_v.at[b]], rows_v.at[b], gsem[b]
                ).wait()
                scale_chunk(b)
                pltpu.async_copy(rows_v.at[b, :, pl.ds(0, D)], out_slice(g), osem[b])
                pltpu.sync_copy(
                    idx_hbm.at[pl.ds(row0 + NBUF * C, C)], idx_v.at[b]
                )
                pltpu.make_async_copy(
                    rows_v.at[b, :, pl.ds(0, D)], out_slice(g), osem[b]
                ).wait()
                pltpu.async_copy(lut_hbm.at[idx_v.at[b]], rows_v.at[b], gsem[b])
            return carry

        lax.fori_loop(0, ngroups - 1, group, 0)

        for b in range(NBUF):
            g = (ngroups - 1) * NBUF + b
            pltpu.make_async_copy(
                lut_hbm.at[idx_v.at[b]], rows_v.at[b], gsem[b]
            ).wait()
            scale_chunk(b)
            pltpu.async_copy(rows_v.at[b, :, pl.ds(0, D)], out_slice(g), osem[b])
        for b in range(NBUF):
            g = (ngroups - 1) * NBUF + b
            pltpu.make_async_copy(rows_v.at[b, :, pl.ds(0, D)], out_slice(g), osem[b]).wait()

    return k


def kernel(x, lut):
    B = x.shape[0] * x.shape[1]
    flat_idx = x.reshape(B)
    lut_p = jnp.pad(lut, ((0, 0), (0, 128 - D_MODEL)))
    out128 = _make_kernel(B, D_MODEL, 400, 2, 8)(flat_idx, lut_p)
    return out128[:, :D_MODEL].reshape(x.shape[0], x.shape[1], D_MODEL)


# speedup vs baseline: 1.1194x; 1.1194x over previous
import functools
import math

import jax
import jax.numpy as jnp
from jax import lax
from jax.experimental import pallas as pl
from jax.experimental.pallas import tpu as pltpu
from jax.experimental.pallas import tpu_sc as plsc

D_MODEL = 64
SCALE = math.sqrt(D_MODEL)

_info = plsc.get_sparse_core_info()
NC, NS, L = _info.num_cores, _info.num_subcores, _info.num_lanes
NW = NC * NS


def _make_kernel(B, D, C, NBUF, U):
    per_w = B // NW
    nchunks = per_w // C
    ngroups = nchunks // NBUF
    assert per_w % C == 0 and nchunks % NBUF == 0 and C % U == 0
    assert (C * D) % 128 == 0
    mesh = plsc.VectorSubcoreMesh(core_axis_name="c", subcore_axis_name="s")

    @functools.partial(
        pl.kernel,
        mesh=mesh,
        out_type=jax.ShapeDtypeStruct((B, 2 * D), jnp.float32),
        scratch_types=[
            pltpu.VMEM((NBUF, C), jnp.int32),
            pltpu.VMEM((NBUF, C), jnp.int32),
            pltpu.VMEM((NBUF, C, D), jnp.float32),
        ]
        + [pltpu.SemaphoreType.DMA] * (2 * NBUF),
        compiler_params=pltpu.CompilerParams(use_tc_tiling_on_sc=False),
    )
    def k(idx_hbm, lut_hbm, out_hbm, idx_v, idx2_v, rows_v, *sems):
        gsem, osem = sems[:NBUF], sems[NBUF:]
        wid = lax.axis_index("s") * NC + lax.axis_index("c")
        base = wid * per_w

        def scale_chunk(b):
            def body(r0, carry):
                for u in range(U):
                    r = r0 * U + u
                    for j in range(D // L):
                        sl = pl.ds(j * L, L)
                        rows_v[b, r, sl] = rows_v[b, r, sl] * SCALE
                return carry

            lax.fori_loop(0, C // U, body, 0)

        def out_slice(g):
            return out_hbm.at[pl.ds(base + g * C, C), pl.ds(0, D)]

        def load_idx(b, row0):
            pltpu.sync_copy(idx_hbm.at[pl.ds(row0, C)], idx_v.at[b])

            def dbl(i, carry):
                sl = pl.ds(i * L, L)
                idx2_v[b, sl] = idx_v[b, sl] * 2
                return carry

            lax.fori_loop(0, C // L, dbl, 0)

        for b in range(NBUF):
            load_idx(b, base + b * C)
            pltpu.async_copy(lut_hbm.at[idx2_v.at[b]], rows_v.at[b], gsem[b])

        def group(gi, carry):
            for b in range(NBUF):
                g = gi * NBUF + b
                row0 = base + g * C
                pltpu.make_async_copy(
                    lut_hbm.at[idx2_v.at[b]], rows_v.at[b], gsem[b]
                ).wait()
                scale_chunk(b)
                pltpu.async_copy(rows_v.at[b], out_slice(g), osem[b])
                load_idx(b, row0 + NBUF * C)
                pltpu.make_async_copy(
                    rows_v.at[b], out_slice(g), osem[b]
                ).wait()
                pltpu.async_copy(lut_hbm.at[idx2_v.at[b]], rows_v.at[b], gsem[b])
            return carry

        lax.fori_loop(0, ngroups - 1, group, 0)

        for b in range(NBUF):
            g = (ngroups - 1) * NBUF + b
            pltpu.make_async_copy(
                lut_hbm.at[idx2_v.at[b]], rows_v.at[b], gsem[b]
            ).wait()
            scale_chunk(b)
            pltpu.async_copy(rows_v.at[b], out_slice(g), osem[b])
        for b in range(NBUF):
            g = (ngroups - 1) * NBUF + b
            pltpu.make_async_copy(rows_v.at[b], out_slice(g), osem[b]).wait()

    return k


def kernel(x, lut):
    B = x.shape[0] * x.shape[1]
    flat_idx = x.reshape(B)
    lut_p = jnp.pad(lut, ((0, 0), (0, 128 - D_MODEL))).reshape(-1, D_MODEL)
    out128 = _make_kernel(B, D_MODEL, 640, 2, 8)(flat_idx, lut_p)
    return out128[:, :D_MODEL].reshape(x.shape[0], x.shape[1], D_MODEL)


# speedup vs baseline: 1.1232x; 1.0034x over previous
import functools
import math

import jax
import jax.numpy as jnp
from jax import lax
from jax.experimental import pallas as pl
from jax.experimental.pallas import tpu as pltpu
from jax.experimental.pallas import tpu_sc as plsc

D_MODEL = 64
SCALE = math.sqrt(D_MODEL)

_info = plsc.get_sparse_core_info()
NC, NS, L = _info.num_cores, _info.num_subcores, _info.num_lanes
NW = NC * NS


def _make_kernel(B, D, C, NBUF, U):
    per_w = B // NW
    nchunks = per_w // C
    ngroups = nchunks // NBUF
    assert per_w % C == 0 and nchunks % NBUF == 0 and C % U == 0
    assert (C * D) % 128 == 0
    mesh = plsc.VectorSubcoreMesh(core_axis_name="c", subcore_axis_name="s")

    @functools.partial(
        pl.kernel,
        mesh=mesh,
        out_type=jax.ShapeDtypeStruct((B, 2 * D), jnp.float32),
        scratch_types=[
            pltpu.VMEM((NBUF, C), jnp.int32),
            pltpu.VMEM((NBUF, C, D), jnp.float32),
        ]
        + [pltpu.SemaphoreType.DMA] * (2 * NBUF),
        compiler_params=pltpu.CompilerParams(use_tc_tiling_on_sc=False),
    )
    def k(idx_hbm, lut_hbm, out_hbm, idx_v, rows_v, *sems):
        gsem, osem = sems[:NBUF], sems[NBUF:]
        wid = lax.axis_index("s") * NC + lax.axis_index("c")
        base = wid * per_w

        def scale_chunk(b):
            def body(r0, carry):
                for u in range(U):
                    r = r0 * U + u
                    for j in range(D // L):
                        sl = pl.ds(j * L, L)
                        rows_v[b, r, sl] = rows_v[b, r, sl] * SCALE
                return carry

            lax.fori_loop(0, C // U, body, 0)

        def out_slice(g):
            return out_hbm.at[pl.ds(base + g * C, C), pl.ds(0, D)]

        def load_idx(b, row0):
            pltpu.sync_copy(idx_hbm.at[pl.ds(row0, C)], idx_v.at[b])

        for b in range(NBUF):
            load_idx(b, base + b * C)
            pltpu.async_copy(lut_hbm.at[idx_v.at[b]], rows_v.at[b], gsem[b])

        def group(gi, carry):
            for b in range(NBUF):
                g = gi * NBUF + b
                row0 = base + g * C
                pltpu.make_async_copy(
                    lut_hbm.at[idx_v.at[b]], rows_v.at[b], gsem[b]
                ).wait()
                scale_chunk(b)
                pltpu.async_copy(rows_v.at[b], out_slice(g), osem[b])
                load_idx(b, row0 + NBUF * C)
                pltpu.make_async_copy(
                    rows_v.at[b], out_slice(g), osem[b]
                ).wait()
                pltpu.async_copy(lut_hbm.at[idx_v.at[b]], rows_v.at[b], gsem[b])
            return carry

        lax.fori_loop(0, ngroups - 1, group, 0)

        for b in range(NBUF):
            g = (ngroups - 1) * NBUF + b
            pltpu.make_async_copy(
                lut_hbm.at[idx_v.at[b]], rows_v.at[b], gsem[b]
            ).wait()
            scale_chunk(b)
            pltpu.async_copy(rows_v.at[b], out_slice(g), osem[b])
        for b in range(NBUF):
            g = (ngroups - 1) * NBUF + b
            pltpu.make_async_copy(rows_v.at[b], out_slice(g), osem[b]).wait()

    return k


def kernel(x, lut):
    B = x.shape[0] * x.shape[1]
    flat_idx2 = x.reshape(B) * 2
    lut_p = jnp.pad(lut, ((0, 0), (0, 128 - D_MODEL))).reshape(-1, D_MODEL)
    out128 = _make_kernel(B, D_MODEL, 400, 4, 8)(flat_idx2, lut_p)
    return out128[:, :D_MODEL].reshape(x.shape[0], x.shape[1], D_MODEL)


# speedup vs baseline: 1.1246x; 1.0013x over previous
import functools
import math

import jax
import jax.numpy as jnp
from jax import lax
from jax.experimental import pallas as pl
from jax.experimental.pallas import tpu as pltpu
from jax.experimental.pallas import tpu_sc as plsc

D_MODEL = 64
SCALE = math.sqrt(D_MODEL)

_info = plsc.get_sparse_core_info()
NC, NS, L = _info.num_cores, _info.num_subcores, _info.num_lanes
NW = NC * NS


def _make_kernel(B, D, C, NBUF, U):
    per_w = B // NW
    nchunks = per_w // C
    ngroups = nchunks // NBUF
    assert per_w % C == 0 and nchunks % NBUF == 0 and C % U == 0
    assert (C * D) % 128 == 0
    mesh = plsc.VectorSubcoreMesh(core_axis_name="c", subcore_axis_name="s")

    @functools.partial(
        pl.kernel,
        mesh=mesh,
        out_type=jax.ShapeDtypeStruct((B, 2 * D), jnp.float32),
        scratch_types=[
            pltpu.VMEM((NBUF, C), jnp.int32),
            pltpu.VMEM((NBUF, C, D), jnp.float32),
        ]
        + [pltpu.SemaphoreType.DMA] * (2 * NBUF),
        compiler_params=pltpu.CompilerParams(use_tc_tiling_on_sc=False),
    )
    def k(idx_hbm, lut_hbm, out_hbm, idx_v, rows_v, *sems):
        gsem, osem = sems[:NBUF], sems[NBUF:]
        wid = lax.axis_index("s") * NC + lax.axis_index("c")
        base = wid * per_w

        def out_slice(g):
            return out_hbm.at[pl.ds(base + g * C, C), pl.ds(0, D)]

        def load_idx(b, row0):
            pltpu.sync_copy(idx_hbm.at[pl.ds(row0, C)], idx_v.at[b])

        for b in range(NBUF):
            load_idx(b, base + b * C)
            pltpu.async_copy(lut_hbm.at[idx_v.at[b]], rows_v.at[b], gsem[b])

        def group(gi, carry):
            for b in range(NBUF):
                g = gi * NBUF + b
                row0 = base + g * C
                pltpu.make_async_copy(
                    lut_hbm.at[idx_v.at[b]], rows_v.at[b], gsem[b]
                ).wait()
                pltpu.async_copy(rows_v.at[b], out_slice(g), osem[b])
                load_idx(b, row0 + NBUF * C)
                pltpu.make_async_copy(
                    rows_v.at[b], out_slice(g), osem[b]
                ).wait()
                pltpu.async_copy(lut_hbm.at[idx_v.at[b]], rows_v.at[b], gsem[b])
            return carry

        lax.fori_loop(0, ngroups - 1, group, 0)

        for b in range(NBUF):
            g = (ngroups - 1) * NBUF + b
            pltpu.make_async_copy(
                lut_hbm.at[idx_v.at[b]], rows_v.at[b], gsem[b]
            ).wait()
            pltpu.async_copy(rows_v.at[b], out_slice(g), osem[b])
        for b in range(NBUF):
            g = (ngroups - 1) * NBUF + b
            pltpu.make_async_copy(rows_v.at[b], out_slice(g), osem[b]).wait()

    return k


def kernel(x, lut):
    B = x.shape[0] * x.shape[1]
    flat_idx2 = x.reshape(B) * 2
    lut_p = (jnp.pad(lut, ((0, 0), (0, 128 - D_MODEL))) * SCALE).reshape(-1, D_MODEL)
    out128 = _make_kernel(B, D_MODEL, 400, 4, 8)(flat_idx2, lut_p)
    return out128[:, :D_MODEL].reshape(x.shape[0], x.shape[1], D_MODEL)
